# baseline (speedup 1.0000x reference)
import jax
import jax.numpy as jnp
from jax import lax
from jax.experimental import pallas as pl
from jax.experimental.pallas import tpu as pltpu
from jax.experimental.pallas import tpu_sc as plsc

B = 4096
D = 1024
F = 65536
K = 32
BF = 256
C = 32
NCH = BF // C
NCHUNK = F // C
NSL = F // 128
NSS = 64

NW = 32
RPW = B // NW
NEG = -3.0e38
CAP = 176
CAP2 = 128


def _mm_body(x_ref, w_ref, s_ref, cm_ref, ss_ref):
    w = w_ref[...]
    norm = jnp.sqrt(jnp.sum(w * w, axis=1, keepdims=True)) + 1e-6
    wn = w / norm
    s = jax.lax.dot_general(
        x_ref[...], wn, (((1,), (1,)), ((), ())),
        preferred_element_type=jnp.float32)
    s_ref[...] = s
    cm = jnp.max(s.reshape(B, NCH, C), axis=-1)
    cm_ref[0] = cm
    ss_ref[0] = jnp.max(cm, axis=-1, keepdims=True)


def _sort16(k):
    return lax.rev(jnp.sort(k), (0,))


def _merge16k(ak, bk):
    rbk = lax.rev(bk, (0,))
    hik = jnp.maximum(ak, rbk)
    lok = jnp.minimum(ak, rbk)
    return _sort16(hik), _sort16(lok)


def _merge16kv(ak, av, bk, bv):
    rbk = lax.rev(bk, (0,))
    rbv = lax.rev(bv, (0,))
    m = ak >= rbk
    hik = jnp.where(m, ak, rbk)
    hiv = jnp.where(m, av, rbv)
    lok = jnp.where(m, rbk, ak)
    lov = jnp.where(m, rbv, av)
    hik, hiv = plsc.sort_key_val(hik, hiv, descending=True)
    lok, lov = plsc.sort_key_val(lok, lov, descending=True)
    return hik, hiv, lok, lov


def _sc_body(srows, cmax, ssm, thr_hbm, cbuf, sbuf, gbuf, candv, candi,
             cand2, thr, dsem, gsem):
    nc = 2
    wid = lax.axis_index("s") * nc + lax.axis_index("c")
    base = wid * RPW
    lanes = lax.iota(jnp.int32, 16)
    lane0 = lanes == 0
    zero16 = jnp.zeros((16,), jnp.int32)
    neg16 = jnp.full((16,), NEG, jnp.float32)

    pltpu.async_copy(cmax.at[base], cbuf.at[0], dsem)
    pltpu.async_copy(ssm.at[base], sbuf.at[0], dsem)

    def row_body(r, _):
        row = base + r
        buf = r & 1
        @pl.when(r + 1 < RPW)
        def _():
            pltpu.async_copy(cmax.at[row + 1], cbuf.at[(r + 1) & 1], dsem)
            pltpu.async_copy(ssm.at[row + 1], sbuf.at[(r + 1) & 1], dsem)
        pltpu.make_async_copy(cmax.at[row], cbuf.at[buf], dsem).wait()
        pltpu.make_async_copy(ssm.at[row], sbuf.at[buf], dsem).wait()

        u0 = _sort16(sbuf[buf, pl.ds(0, 16)])
        u1 = neg16

        def ssmerge(j, carry):
            u0, u1 = carry
            n = _sort16(sbuf[buf, pl.ds(j * 16, 16)])
            h, _ = _merge16k(u1, n)
            return _merge16k(u0, h)

        u0, u1 = lax.fori_loop(1, NSS // 16, ssmerge, (u0, u1))
        ss32 = jnp.min(u1)

        for j in range(CAP // 16):
            candv[pl.ds(j * 16, 16)] = neg16
            candi[pl.ds(j * 16, 16)] = zero16

        def ascan(i, cnt):
            for u in range(4):
                off = (i * 4 + u) * 16
                v = cbuf[buf, pl.ds(off, 16)]
                m = v >= ss32
                mi = m.astype(jnp.int32)
                excl = plsc.cumsum(mi) - mi
                idxs = jnp.minimum(cnt + excl, CAP - 1)
                plsc.store_scatter(candv, [idxs], v, mask=m)
                plsc.store_scatter(candi, [idxs], lanes + off, mask=m)
                cnt = cnt + plsc.all_reduce_population_count(m)
            return cnt

        lax.fori_loop(0, NCHUNK // 64, ascan, zero16)

        t0k, t0v = plsc.sort_key_val(candv[pl.ds(0, 16)],
                                     candi[pl.ds(0, 16)], descending=True)
        t1k, t1v = neg16, zero16

        def amerge(j, carry):
            t0k, t0v, t1k, t1v = carry
            nk, nv = plsc.sort_key_val(candv[pl.ds(j * 16, 16)],
                                       candi[pl.ds(j * 16, 16)],
                                       descending=True)
            hk, hv, _, _ = _merge16kv(t1k, t1v, nk, nv)
            return _merge16kv(t0k, t0v, hk, hv)

        t0k, t0v, t1k, t1v = lax.fori_loop(1, CAP // 16, amerge,
                                           (t0k, t0v, t1k, t1v))
        tau = jnp.min(t1k)

        rbase = row * NSL
        s0 = jnp.clip(t0v >> 2, 0, NSL - 1) + rbase
        s1 = jnp.clip(t1v >> 2, 0, NSL - 1) + rbase
        d0 = pltpu.async_copy(srows.at[s0], gbuf.at[pl.ds(0, 16)], gsem)
        d1 = pltpu.async_copy(srows.at[s1], gbuf.at[pl.ds(16, 16)], gsem)
        for j in range(CAP2 // 16):
            cand2[pl.ds(j * 16, 16)] = neg16
        d0.wait()
        d1.wait()

        cnt2 = zero16
        for half, gidv in ((0, t0v), (1, t1v)):
            rows16 = lanes + half * 16
            colbase = (gidv & 3) * C

            def cscan(k, cnt, rows16=rows16, colbase=colbase):
                v = plsc.load_gather(gbuf, [rows16, colbase + k])
                m = v >= tau
                mi = m.astype(jnp.int32)
                excl = plsc.cumsum(mi) - mi
                idxs = jnp.minimum(cnt + excl, CAP2 - 1)
                plsc.store_scatter(cand2, [idxs], v, mask=m)
                return cnt + plsc.all_reduce_population_count(m)

            cnt2 = lax.fori_loop(0, C, cscan, cnt2)

        w0 = _sort16(cand2[pl.ds(0, 16)])
        w1 = neg16

        def dmerge(j, carry):
            w0, w1 = carry
            n = _sort16(cand2[pl.ds(j * 16, 16)])
            h, _ = _merge16k(w1, n)
            return _merge16k(w0, h)

        w0, w1 = lax.fori_loop(1, CAP2 // 16, dmerge, (w0, w1))
        t = jnp.min(w1)
        plsc.store_scatter(thr, [jnp.full((16,), r, jnp.int32)],
                           jnp.full((16,), t, jnp.float32), mask=lane0)
        return 0

    lax.fori_loop(0, RPW, row_body, 0)
    pltpu.sync_copy(thr, thr_hbm.at[pl.ds(base, RPW)])


def _mask_body(s_ref, t_ref, o_ref):
    s = s_ref[...]
    o_ref[...] = jnp.where(s >= t_ref[...], jnp.maximum(s, 0.0), 0.0)


def kernel(x, dict_w):
    scores, cmax3, ss3 = pl.pallas_call(
        _mm_body,
        grid=(F // BF,),
        in_specs=[pl.BlockSpec((B, D), lambda j: (0, 0)),
                  pl.BlockSpec((BF, D), lambda j: (j, 0))],
        out_specs=[pl.BlockSpec((B, BF), lambda j: (0, j)),
                   pl.BlockSpec((1, B, NCH), lambda j: (j, 0, 0)),
                   pl.BlockSpec((1, B, 1), lambda j: (j, 0, 0))],
        out_shape=[jax.ShapeDtypeStruct((B, F), jnp.float32),
                   jax.ShapeDtypeStruct((F // BF, B, NCH), jnp.float32),
                   jax.ShapeDtypeStruct((F // BF, B, 1), jnp.float32)],
    )(x, dict_w)

    cmax = cmax3.transpose(1, 0, 2).reshape(B, NCHUNK)
    ssm = jnp.max(ss3[:, :, 0].T.reshape(B, NSS, (F // BF) // NSS), axis=-1)
    srows = scores.reshape(B * NSL, 128)

    mesh = plsc.VectorSubcoreMesh(core_axis_name="c", subcore_axis_name="s")
    thresh = pl.kernel(
        _sc_body,
        out_type=jax.ShapeDtypeStruct((B,), jnp.float32),
        mesh=mesh,
        compiler_params=pltpu.CompilerParams(needs_layout_passes=False),
        scratch_types=[
            pltpu.VMEM((2, NCHUNK), jnp.float32),
            pltpu.VMEM((2, NSS), jnp.float32),
            pltpu.VMEM((K, 128), jnp.float32),
            pltpu.VMEM((CAP,), jnp.float32),
            pltpu.VMEM((CAP,), jnp.int32),
            pltpu.VMEM((CAP2,), jnp.float32),
            pltpu.VMEM((RPW,), jnp.float32),
            pltpu.SemaphoreType.DMA,
            pltpu.SemaphoreType.DMA,
        ],
    )(srows, cmax, ssm)

    return pl.pallas_call(
        _mask_body,
        grid=(F // BF,),
        in_specs=[pl.BlockSpec((B, BF), lambda j: (0, j)),
                  pl.BlockSpec((B, 1), lambda j: (0, 0))],
        out_specs=pl.BlockSpec((B, BF), lambda j: (0, j)),
        out_shape=jax.ShapeDtypeStruct((B, F), jnp.float32),
    )(scores, thresh.reshape(B, 1))


# speedup vs baseline: 1.4290x; 1.4290x over previous
import jax
import jax.numpy as jnp
from jax import lax
from jax.experimental import pallas as pl
from jax.experimental.pallas import tpu as pltpu
from jax.experimental.pallas import tpu_sc as plsc

B = 4096
D = 1024
F = 65536
K = 32
BF = 512
C = 128
NCH = BF // C
NCHUNK = F // C
NSS = 64

NW = 32
RPW = B // NW
NEG = -3.0e38
CAP = 80
CAP2 = 96


def _mm_body(x_ref, w_ref, s_ref, cm_ref):
    w = w_ref[...]
    norm = jnp.sqrt(jnp.sum(w * w, axis=1, keepdims=True)) + 1e-6
    wn = w / norm
    s = jax.lax.dot_general(
        x_ref[...], wn, (((1,), (1,)), ((), ())),
        preferred_element_type=jnp.float32)
    s_ref[...] = s
    cm_ref[0] = jnp.max(s.reshape(B, NCH, C), axis=-1)


def _sort16(k):
    return lax.rev(jnp.sort(k), (0,))


def _merge16k(ak, bk):
    rbk = lax.rev(bk, (0,))
    hik = jnp.maximum(ak, rbk)
    lok = jnp.minimum(ak, rbk)
    return _sort16(hik), _sort16(lok)


def _merge16kv(ak, av, bk, bv):
    rbk = lax.rev(bk, (0,))
    rbv = lax.rev(bv, (0,))
    m = ak >= rbk
    hik = jnp.where(m, ak, rbk)
    hiv = jnp.where(m, av, rbv)
    lok = jnp.where(m, rbk, ak)
    lov = jnp.where(m, rbv, av)
    hik, hiv = plsc.sort_key_val(hik, hiv, descending=True)
    lok, lov = plsc.sort_key_val(lok, lov, descending=True)
    return hik, hiv, lok, lov


def _sc_body(srows, cmax, ssm, thr_hbm, cbuf, sbuf, gbuf, candv, candi,
             cand2, thr, dsem, gsem):
    nc = 2
    wid = lax.axis_index("s") * nc + lax.axis_index("c")
    base = wid * RPW
    lanes = lax.iota(jnp.int32, 16)
    lane0 = lanes == 0
    zero16 = jnp.zeros((16,), jnp.int32)
    neg16 = jnp.full((16,), NEG, jnp.float32)

    pltpu.async_copy(cmax.at[base], cbuf.at[0], dsem)
    pltpu.async_copy(ssm.at[base], sbuf.at[0], dsem)

    def row_body(r, _):
        row = base + r
        buf = r & 1
        @pl.when(r + 1 < RPW)
        def _():
            pltpu.async_copy(cmax.at[row + 1], cbuf.at[(r + 1) & 1], dsem)
            pltpu.async_copy(ssm.at[row + 1], sbuf.at[(r + 1) & 1], dsem)
        pltpu.make_async_copy(cmax.at[row], cbuf.at[buf], dsem).wait()
        pltpu.make_async_copy(ssm.at[row], sbuf.at[buf], dsem).wait()

        u0 = _sort16(sbuf[buf, pl.ds(0, 16)])
        u1 = neg16

        def ssmerge(j, carry):
            u0, u1 = carry
            n = _sort16(sbuf[buf, pl.ds(j * 16, 16)])
            h, _ = _merge16k(u1, n)
            return _merge16k(u0, h)

        u0, u1 = lax.fori_loop(1, NSS // 16, ssmerge, (u0, u1))
        ss32 = jnp.min(u1)

        for j in range(CAP // 16):
            candv[pl.ds(j * 16, 16)] = neg16
            candi[pl.ds(j * 16, 16)] = zero16

        def ascan(i, cnt):
            for u in range(4):
                off = (i * 4 + u) * 16
                v = cbuf[buf, pl.ds(off, 16)]
                m = v >= ss32
                mi = m.astype(jnp.int32)
                excl = plsc.cumsum(mi) - mi
                idxs = jnp.minimum(cnt + excl, CAP - 1)
                plsc.store_scatter(candv, [idxs], v, mask=m)
                plsc.store_scatter(candi, [idxs], lanes + off, mask=m)
                cnt = cnt + plsc.all_reduce_population_count(m)
            return cnt

        cnt = lax.fori_loop(0, NCHUNK // 64, ascan, zero16)
        cnta = jnp.max(cnt)

        t0k, t0v = plsc.sort_key_val(candv[pl.ds(0, 16)],
                                     candi[pl.ds(0, 16)], descending=True)
        t1k, t1v = neg16, zero16

        def amerge(j, carry):
            t0k, t0v, t1k, t1v = carry
            nk, nv = plsc.sort_key_val(candv[pl.ds(j * 16, 16)],
                                       candi[pl.ds(j * 16, 16)],
                                       descending=True)
            hk, hv, _, _ = _merge16kv(t1k, t1v, nk, nv)
            return _merge16kv(t0k, t0v, hk, hv)

        nva = jnp.clip((cnta + 15) >> 4, 2, CAP // 16)
        t0k, t0v, t1k, t1v = lax.fori_loop(1, nva, amerge,
                                           (t0k, t0v, t1k, t1v))
        tau = jnp.min(t1k)

        rbase = row * NCHUNK
        s0 = jnp.clip(t0v, 0, NCHUNK - 1) + rbase
        s1 = jnp.clip(t1v, 0, NCHUNK - 1) + rbase
        d0 = pltpu.async_copy(srows.at[s0], gbuf.at[pl.ds(0, 16)], gsem)
        d1 = pltpu.async_copy(srows.at[s1], gbuf.at[pl.ds(16, 16)], gsem)
        for j in range(CAP2 // 16):
            cand2[pl.ds(j * 16, 16)] = neg16
        d0.wait()
        d1.wait()

        def cscan(i, cnt):
            for u in range(4):
                s = i * 4 + u
                v = gbuf[s >> 3, pl.ds((s & 7) * 16, 16)]
                m = v >= tau
                mi = m.astype(jnp.int32)
                excl = plsc.cumsum(mi) - mi
                idxs = jnp.minimum(cnt + excl, CAP2 - 1)
                plsc.store_scatter(cand2, [idxs], v, mask=m)
                cnt = cnt + plsc.all_reduce_population_count(m)
            return cnt

        cnt2 = lax.fori_loop(0, K * (C // 16) // 4, cscan, zero16)
        cnt2a = jnp.max(cnt2)

        w0 = _sort16(cand2[pl.ds(0, 16)])
        w1 = neg16

        def dmerge(j, carry):
            w0, w1 = carry
            n = _sort16(cand2[pl.ds(j * 16, 16)])
            h, _ = _merge16k(w1, n)
            return _merge16k(w0, h)

        nvd = jnp.clip((cnt2a + 15) >> 4, 2, CAP2 // 16)
        w0, w1 = lax.fori_loop(1, nvd, dmerge, (w0, w1))
        t = jnp.min(w1)
        plsc.store_scatter(thr, [jnp.full((16,), r, jnp.int32)],
                           jnp.full((16,), t, jnp.float32), mask=lane0)
        return 0

    lax.fori_loop(0, RPW, row_body, 0)
    pltpu.sync_copy(thr, thr_hbm.at[pl.ds(base, RPW)])


def _mask_body(s_ref, t_ref, o_ref):
    s = s_ref[...]
    o_ref[...] = jnp.where(s >= t_ref[...], jnp.maximum(s, 0.0), 0.0)


def kernel(x, dict_w):
    scores, cmax3 = pl.pallas_call(
        _mm_body,
        grid=(F // BF,),
        in_specs=[pl.BlockSpec((B, D), lambda j: (0, 0)),
                  pl.BlockSpec((BF, D), lambda j: (j, 0))],
        out_specs=[pl.BlockSpec((B, BF), lambda j: (0, j)),
                   pl.BlockSpec((1, B, NCH), lambda j: (j, 0, 0))],
        out_shape=[jax.ShapeDtypeStruct((B, F), jnp.float32),
                   jax.ShapeDtypeStruct((F // BF, B, NCH), jnp.float32)],
    )(x, dict_w)

    cmax = cmax3.transpose(1, 0, 2).reshape(B, NCHUNK)
    ssm = jnp.max(cmax.reshape(B, NSS, NCHUNK // NSS), axis=-1)
    srows = scores.reshape(B * NCHUNK, C)

    mesh = plsc.VectorSubcoreMesh(core_axis_name="c", subcore_axis_name="s")
    thresh = pl.kernel(
        _sc_body,
        out_type=jax.ShapeDtypeStruct((B,), jnp.float32),
        mesh=mesh,
        compiler_params=pltpu.CompilerParams(needs_layout_passes=False),
        scratch_types=[
            pltpu.VMEM((2, NCHUNK), jnp.float32),
            pltpu.VMEM((2, NSS), jnp.float32),
            pltpu.VMEM((K, C), jnp.float32),
            pltpu.VMEM((CAP,), jnp.float32),
            pltpu.VMEM((CAP,), jnp.int32),
            pltpu.VMEM((CAP2,), jnp.float32),
            pltpu.VMEM((RPW,), jnp.float32),
            pltpu.SemaphoreType.DMA,
            pltpu.SemaphoreType.DMA,
        ],
    )(srows, cmax, ssm)

    return pl.pallas_call(
        _mask_body,
        grid=(F // BF,),
        in_specs=[pl.BlockSpec((B, BF), lambda j: (0, j)),
                  pl.BlockSpec((B, 1), lambda j: (0, 0))],
        out_specs=pl.BlockSpec((B, BF), lambda j: (0, j)),
        out_shape=jax.ShapeDtypeStruct((B, F), jnp.float32),
    )(scores, thresh.reshape(B, 1))
